# baseline (speedup 1.0000x reference)
import functools

import jax
import jax.numpy as jnp
from jax import lax
from jax.experimental import pallas as pl
from jax.experimental.pallas import tpu as pltpu
from jax.experimental.pallas import tpu_sc as plsc

EPS = 1e-12
SC_CORES = 2
SC_SUBCORES = 16
NUM_TILES = SC_CORES * SC_SUBCORES


def _sc_gather(table, idx_flat):
    n = idx_flat.shape[0]
    e = table.shape[1]
    per_tile = n // NUM_TILES
    mesh = plsc.VectorSubcoreMesh(core_axis_name="c", subcore_axis_name="s")

    nchunk = 8
    ck = per_tile // nchunk

    @functools.partial(
        pl.kernel,
        mesh=mesh,
        out_type=jax.ShapeDtypeStruct((n, e), jnp.float32),
        scratch_types=[pltpu.VMEM((per_tile,), jnp.int32)]
        + [pltpu.VMEM((ck, e), jnp.float32) for _ in range(nchunk)]
        + [pltpu.SemaphoreType.DMA for _ in range(nchunk + 1)],
    )
    def gather_kernel(table_hbm, idx_hbm, out_hbm, idx_v, *bufs_sems):
        bufs = bufs_sems[:nchunk]
        gsem = bufs_sems[nchunk]
        wsems = bufs_sems[nchunk + 1:]
        wid = lax.axis_index("s") * SC_CORES + lax.axis_index("c")
        base = wid * per_tile
        pltpu.sync_copy(idx_hbm.at[pl.ds(base, per_tile)], idx_v)
        gathers = [
            pltpu.async_copy(
                table_hbm.at[idx_v.at[pl.ds(k * ck, ck)]], bufs[k], gsem
            )
            for k in range(nchunk)
        ]
        writes = []
        for k in range(nchunk):
            gathers[k].wait()
            writes.append(
                pltpu.async_copy(
                    bufs[k], out_hbm.at[pl.ds(base + k * ck, ck)], wsems[k]
                )
            )
        for wcopy in writes:
            wcopy.wait()

    return gather_kernel(table, idx_flat)


def _tc_combine(gathered, seg_ids_col, pos_emb, seg_pad, ln_w, ln_b, b, s):
    e = gathered.shape[1]
    tblk = 2048
    js = s // tblk

    def body(gw_ref, sid_ref, pos_ref, seg_ref, w_ref, b_ref, o_ref):
        x = gw_ref[...]
        sid = sid_ref[...]
        seg = jnp.where(sid == 0, seg_ref[0:1, :], seg_ref[1:2, :])
        x = x + seg + pos_ref[...]
        u = jnp.mean(x, axis=-1, keepdims=True)
        u2 = jnp.mean(x * x, axis=-1, keepdims=True)
        v = u2 - u * u
        o = (x - u) * lax.rsqrt(v + EPS)
        o_ref[0] = o * w_ref[...] + b_ref[...]

    out = pl.pallas_call(
        body,
        grid=(js, b),
        in_specs=[
            pl.BlockSpec((tblk, e), lambda j, i: (i * js + j, 0)),
            pl.BlockSpec((tblk, 1), lambda j, i: (i * js + j, 0)),
            pl.BlockSpec((tblk, e), lambda j, i: (j, 0)),
            pl.BlockSpec((8, e), lambda j, i: (0, 0)),
            pl.BlockSpec((1, e), lambda j, i: (0, 0)),
            pl.BlockSpec((1, e), lambda j, i: (0, 0)),
        ],
        out_specs=pl.BlockSpec(
            (1, tblk, e), lambda j, i: (i, j, 0)
        ),
        out_shape=jax.ShapeDtypeStruct((b, s, e), jnp.float32),
    )(gathered, seg_ids_col, pos_emb, seg_pad, ln_w, ln_b)
    return out


def kernel(token_ids, segment_ids, word_emb, seg_emb, pos_emb, ln_weight, ln_bias):
    b, s = token_ids.shape
    e = word_emb.shape[1]
    idx_flat = token_ids.astype(jnp.int32).reshape(b * s)
    gathered = _sc_gather(word_emb, idx_flat)
    seg_ids_col = segment_ids.astype(jnp.int32).reshape(b * s, 1)
    seg_pad = jnp.zeros((8, e), jnp.float32).at[: seg_emb.shape[0]].set(seg_emb)
    pos = pos_emb[:s]
    return _tc_combine(
        gathered,
        seg_ids_col,
        pos,
        seg_pad,
        ln_weight.reshape(1, e),
        ln_bias.reshape(1, e),
        b,
        s,
    )


# speedup vs baseline: 1.0057x; 1.0057x over previous
import functools

import jax
import jax.numpy as jnp
from jax import lax
from jax.experimental import pallas as pl
from jax.experimental.pallas import tpu as pltpu
from jax.experimental.pallas import tpu_sc as plsc

EPS = 1e-12
SC_CORES = 2
SC_SUBCORES = 16
NUM_TILES = SC_CORES * SC_SUBCORES


def _sc_gather(table, idx_flat):
    n = idx_flat.shape[0]
    e = table.shape[1]
    per_tile = n // NUM_TILES
    mesh = plsc.VectorSubcoreMesh(core_axis_name="c", subcore_axis_name="s")

    nchunk = 4
    ck = per_tile // nchunk

    @functools.partial(
        pl.kernel,
        mesh=mesh,
        out_type=jax.ShapeDtypeStruct((n, e), jnp.float32),
        scratch_types=[pltpu.VMEM((per_tile,), jnp.int32)]
        + [pltpu.VMEM((ck, e), jnp.float32) for _ in range(nchunk)]
        + [pltpu.SemaphoreType.DMA for _ in range(nchunk + 1)],
    )
    def gather_kernel(table_hbm, idx_hbm, out_hbm, idx_v, *bufs_sems):
        bufs = bufs_sems[:nchunk]
        gsem = bufs_sems[nchunk]
        wsems = bufs_sems[nchunk + 1:]
        wid = lax.axis_index("s") * SC_CORES + lax.axis_index("c")
        base = wid * per_tile
        pltpu.sync_copy(idx_hbm.at[pl.ds(base, per_tile)], idx_v)
        gathers = [
            pltpu.async_copy(
                table_hbm.at[idx_v.at[pl.ds(k * ck, ck)]], bufs[k], gsem
            )
            for k in range(nchunk)
        ]
        writes = []
        for k in range(nchunk):
            gathers[k].wait()
            writes.append(
                pltpu.async_copy(
                    bufs[k], out_hbm.at[pl.ds(base + k * ck, ck)], wsems[k]
                )
            )
        for wcopy in writes:
            wcopy.wait()

    return gather_kernel(table, idx_flat)


def _tc_combine(gathered, seg_ids_col, pos_emb, seg_pad, ln_w, ln_b, b, s):
    e = gathered.shape[1]
    tblk = 2048
    js = s // tblk

    def body(gw_ref, sid_ref, pos_ref, seg_ref, w_ref, b_ref, o_ref):
        x = gw_ref[...]
        sid = sid_ref[...]
        seg = jnp.where(sid == 0, seg_ref[0:1, :], seg_ref[1:2, :])
        x = x + seg + pos_ref[...]
        u = jnp.mean(x, axis=-1, keepdims=True)
        u2 = jnp.mean(x * x, axis=-1, keepdims=True)
        v = u2 - u * u
        o = (x - u) * lax.rsqrt(v + EPS)
        o_ref[0] = o * w_ref[...] + b_ref[...]

    out = pl.pallas_call(
        body,
        grid=(js, b),
        in_specs=[
            pl.BlockSpec((tblk, e), lambda j, i: (i * js + j, 0)),
            pl.BlockSpec((tblk, 1), lambda j, i: (i * js + j, 0)),
            pl.BlockSpec((tblk, e), lambda j, i: (j, 0)),
            pl.BlockSpec((8, e), lambda j, i: (0, 0)),
            pl.BlockSpec((1, e), lambda j, i: (0, 0)),
            pl.BlockSpec((1, e), lambda j, i: (0, 0)),
        ],
        out_specs=pl.BlockSpec(
            (1, tblk, e), lambda j, i: (i, j, 0)
        ),
        out_shape=jax.ShapeDtypeStruct((b, s, e), jnp.float32),
    )(gathered, seg_ids_col, pos_emb, seg_pad, ln_w, ln_b)
    return out


def kernel(token_ids, segment_ids, word_emb, seg_emb, pos_emb, ln_weight, ln_bias):
    b, s = token_ids.shape
    e = word_emb.shape[1]
    idx_flat = token_ids.astype(jnp.int32).reshape(b * s)
    gathered = _sc_gather(word_emb, idx_flat)
    seg_ids_col = segment_ids.astype(jnp.int32).reshape(b * s, 1)
    seg_pad = jnp.zeros((8, e), jnp.float32).at[: seg_emb.shape[0]].set(seg_emb)
    pos = pos_emb[:s]
    return _tc_combine(
        gathered,
        seg_ids_col,
        pos,
        seg_pad,
        ln_weight.reshape(1, e),
        ln_bias.reshape(1, e),
        b,
        s,
    )


# speedup vs baseline: 1.0068x; 1.0010x over previous
import functools

import jax
import jax.numpy as jnp
from jax import lax
from jax.experimental import pallas as pl
from jax.experimental.pallas import tpu as pltpu
from jax.experimental.pallas import tpu_sc as plsc

EPS = 1e-12
SC_CORES = 2
SC_SUBCORES = 16
NUM_TILES = SC_CORES * SC_SUBCORES


def _sc_gather(table, idx_flat):
    n = idx_flat.shape[0]
    e = table.shape[1]
    per_tile = n // NUM_TILES
    mesh = plsc.VectorSubcoreMesh(core_axis_name="c", subcore_axis_name="s")

    nchunk = 4
    ck = per_tile // nchunk

    @functools.partial(
        pl.kernel,
        mesh=mesh,
        out_type=jax.ShapeDtypeStruct((n, e), jnp.float32),
        scratch_types=[pltpu.VMEM((per_tile,), jnp.int32)]
        + [pltpu.VMEM((ck, e), jnp.float32) for _ in range(nchunk)]
        + [pltpu.SemaphoreType.DMA for _ in range(nchunk + 1)],
    )
    def gather_kernel(table_hbm, idx_hbm, out_hbm, idx_v, *bufs_sems):
        bufs = bufs_sems[:nchunk]
        gsem = bufs_sems[nchunk]
        wsems = bufs_sems[nchunk + 1:]
        wid = lax.axis_index("s") * SC_CORES + lax.axis_index("c")
        base = wid * per_tile
        pltpu.sync_copy(idx_hbm.at[pl.ds(base, per_tile)], idx_v)
        gathers = [
            pltpu.async_copy(
                table_hbm.at[idx_v.at[pl.ds(k * ck, ck)]], bufs[k], gsem
            )
            for k in range(nchunk)
        ]
        writes = []
        for k in range(nchunk):
            gathers[k].wait()
            writes.append(
                pltpu.async_copy(
                    bufs[k], out_hbm.at[pl.ds(base + k * ck, ck)], wsems[k]
                )
            )
        for wcopy in writes:
            wcopy.wait()

    return gather_kernel(table, idx_flat)


def _tc_combine(gathered, seg_ids_col, pos_emb, seg_pad, ln_w, ln_b, b, s):
    e = gathered.shape[1]
    tblk = 2048
    js = s // tblk

    def body(gw_ref, sid_ref, pos_ref, seg_ref, w_ref, b_ref, o_ref):
        x = gw_ref[...]
        sid = sid_ref[...]
        seg = jnp.where(sid == 0, seg_ref[0:1, :], seg_ref[1:2, :])
        x = x + seg + pos_ref[...]
        u = jnp.mean(x, axis=-1, keepdims=True)
        u2 = jnp.mean(x * x, axis=-1, keepdims=True)
        v = u2 - u * u
        o = (x - u) * lax.rsqrt(v + EPS)
        o_ref[0] = o * w_ref[...] + b_ref[...]

    out = pl.pallas_call(
        body,
        grid=(js, b),
        compiler_params=pltpu.CompilerParams(
            dimension_semantics=("parallel", "parallel")
        ),
        in_specs=[
            pl.BlockSpec((tblk, e), lambda j, i: (i * js + j, 0)),
            pl.BlockSpec((tblk, 1), lambda j, i: (i * js + j, 0)),
            pl.BlockSpec((tblk, e), lambda j, i: (j, 0)),
            pl.BlockSpec((8, e), lambda j, i: (0, 0)),
            pl.BlockSpec((1, e), lambda j, i: (0, 0)),
            pl.BlockSpec((1, e), lambda j, i: (0, 0)),
        ],
        out_specs=pl.BlockSpec(
            (1, tblk, e), lambda j, i: (i, j, 0)
        ),
        out_shape=jax.ShapeDtypeStruct((b, s, e), jnp.float32),
    )(gathered, seg_ids_col, pos_emb, seg_pad, ln_w, ln_b)
    return out


def kernel(token_ids, segment_ids, word_emb, seg_emb, pos_emb, ln_weight, ln_bias):
    b, s = token_ids.shape
    e = word_emb.shape[1]
    idx_flat = token_ids.astype(jnp.int32).reshape(b * s)
    gathered = _sc_gather(word_emb, idx_flat)
    seg_ids_col = segment_ids.astype(jnp.int32).reshape(b * s, 1)
    seg_pad = jnp.zeros((8, e), jnp.float32).at[: seg_emb.shape[0]].set(seg_emb)
    pos = pos_emb[:s]
    return _tc_combine(
        gathered,
        seg_ids_col,
        pos,
        seg_pad,
        ln_weight.reshape(1, e),
        ln_bias.reshape(1, e),
        b,
        s,
    )
